# baseline (device time: 71815 ns/iter reference)
import jax
import jax.numpy as jnp
from jax import lax
from jax.experimental import pallas as pl
from jax.experimental.pallas import tpu as pltpu

N_DEV = 4
SQ = 1024
SKV = 1024
H_LOC = 8
DH = 128
D_MODEL = 1024
WINDOW = 128
SCALE = 0.08838834764831843
BLK = 256


def kernel(x, Wq, K_ext, V_ext, Wo):
    def body(x_ref, wq_ref, k_hbm, v_hbm, wo_ref, out_ref,
             k_vmem, v_vmem, q_ref, ctx_ref, ar_ref, comm1_ref, comm2_ref,
             send_sems, recv_sems, cp_sems):
        my = lax.axis_index("i")
        head0 = my * H_LOC

        p1 = my ^ 1
        p2 = 3 - my
        h1 = ((my == 1) | (my == 2)).astype(jnp.int32)
        h2 = my // 2
        g1 = my // 2
        g2 = my % 2
        a_keep = h1 * 256
        a_send = (1 - h1) * 256
        b_keep = 512 + g1 * 256
        b_send = 512 + (1 - g1) * 256
        qa = h1 * 256 + h2 * 128
        qb = 512 + g1 * 256 + g2 * 128

        barrier = pltpu.get_barrier_semaphore()
        for nbr in [p1, p2]:
            pl.semaphore_signal(
                barrier, inc=1,
                device_id=(nbr,), device_id_type=pl.DeviceIdType.MESH)
        pl.semaphore_wait(barrier, 2)

        cp_k = pltpu.make_async_copy(
            k_hbm.at[0, :, pl.ds(head0, H_LOC), :], k_vmem, cp_sems.at[0])
        cp_v = pltpu.make_async_copy(
            v_hbm.at[0, :, pl.ds(head0, H_LOC), :], v_vmem, cp_sems.at[1])
        cp_k.start()
        cp_v.start()

        q_ref[...] = jnp.dot(x_ref[0], wq_ref[...],
                             preferred_element_type=jnp.float32)

        cp_k.wait()
        cp_v.wait()

        neg = jnp.float32(-1e9)
        iota_q = lax.broadcasted_iota(jnp.int32, (BLK, SKV), 0)
        iota_k = lax.broadcasted_iota(jnp.int32, (BLK, SKV), 1)

        def compute_block(row0):
            mask = jnp.abs(iota_q + row0 - iota_k) <= WINDOW
            rows = pl.ds(row0, BLK)
            for h in range(H_LOC):
                qh = q_ref[rows, h * DH:(h + 1) * DH]
                kh = k_vmem[:, h, :]
                s = lax.dot_general(
                    qh, kh, (((1,), (1,)), ((), ())),
                    preferred_element_type=jnp.float32) * SCALE
                s = jnp.where(mask, s, neg)
                m = jnp.max(s, axis=1, keepdims=True)
                w = jnp.exp(s - m)
                w = w / jnp.sum(w, axis=1, keepdims=True)
                ctx_ref[rows, h * DH:(h + 1) * DH] = jnp.dot(
                    w, v_vmem[:, h, :], preferred_element_type=jnp.float32)
            ar_ref[rows, :] = jnp.dot(
                ctx_ref[rows, :], wo_ref[...],
                preferred_element_type=jnp.float32).astype(jnp.bfloat16)

        def exch(src_rows, n_rows, dst_ref, dst_rows, sem_idx, peer):
            return pltpu.make_async_remote_copy(
                src_ref=ar_ref.at[pl.ds(src_rows, n_rows), :],
                dst_ref=dst_ref.at[pl.ds(dst_rows, n_rows), :],
                send_sem=send_sems.at[sem_idx],
                recv_sem=recv_sems.at[sem_idx],
                device_id=(peer,),
                device_id_type=pl.DeviceIdType.MESH)

        compute_block(a_send)
        ra = exch(a_send, 256, comm1_ref, 0, 0, p1)
        ra.start()
        compute_block(b_send)
        rb = exch(b_send, 256, comm1_ref, 256, 1, p2)
        rb.start()
        compute_block(a_keep)

        ra.wait()
        rows = pl.ds(a_keep, 256)
        ar_ref[rows, :] = ar_ref[rows, :] + comm1_ref[pl.ds(0, 256), :]
        ra = exch(a_keep + (1 - h2) * 128, 128, comm2_ref, 0, 2, p2)
        ra.start()

        compute_block(b_keep)

        rb.wait()
        rows = pl.ds(b_keep, 256)
        ar_ref[rows, :] = ar_ref[rows, :] + comm1_ref[pl.ds(256, 256), :]
        rb = exch(b_keep + (1 - g2) * 128, 128, comm2_ref, 128, 3, p1)
        rb.start()

        ra.wait()
        rows = pl.ds(qa, 128)
        ar_ref[rows, :] = ar_ref[rows, :] + comm2_ref[pl.ds(0, 128), :]
        ra = exch(qa, 128, ar_ref, qa, 4, p2)
        ra.start()

        rb.wait()
        rows = pl.ds(qb, 128)
        ar_ref[rows, :] = ar_ref[rows, :] + comm2_ref[pl.ds(128, 128), :]
        rb = exch(qb, 128, ar_ref, qb, 5, p1)
        rb.start()

        ra.wait()
        ra = exch(a_keep, 256, ar_ref, a_keep, 6, p1)
        ra.start()
        rb.wait()
        rb = exch(b_keep, 256, ar_ref, b_keep, 7, p2)
        rb.start()
        ra.wait()
        rb.wait()

        out_ref[0] = ar_ref[...].astype(jnp.float32)

    return pl.pallas_call(
        body,
        out_shape=jax.ShapeDtypeStruct((1, SQ, D_MODEL), jnp.float32),
        in_specs=[
            pl.BlockSpec(memory_space=pltpu.VMEM),
            pl.BlockSpec(memory_space=pltpu.VMEM),
            pl.BlockSpec(memory_space=pl.ANY),
            pl.BlockSpec(memory_space=pl.ANY),
            pl.BlockSpec(memory_space=pltpu.VMEM),
        ],
        out_specs=pl.BlockSpec(memory_space=pltpu.VMEM),
        scratch_shapes=[
            pltpu.VMEM((SKV, H_LOC, DH), jnp.float32),
            pltpu.VMEM((SKV, H_LOC, DH), jnp.float32),
            pltpu.VMEM((SQ, D_MODEL), jnp.float32),
            pltpu.VMEM((SQ, D_MODEL), jnp.float32),
            pltpu.VMEM((SQ, D_MODEL), jnp.bfloat16),
            pltpu.VMEM((512, D_MODEL), jnp.bfloat16),
            pltpu.VMEM((256, D_MODEL), jnp.bfloat16),
            pltpu.SemaphoreType.DMA((8,)),
            pltpu.SemaphoreType.DMA((8,)),
            pltpu.SemaphoreType.DMA((2,)),
        ],
        compiler_params=pltpu.CompilerParams(collective_id=0),
    )(x, Wq, K_ext, V_ext, Wo)


# device time: 68866 ns/iter; 1.0428x vs baseline; 1.0428x over previous
import jax
import jax.numpy as jnp
from jax import lax
from jax.experimental import pallas as pl
from jax.experimental.pallas import tpu as pltpu

N_DEV = 4
SQ = 1024
SKV = 1024
H_LOC = 8
DH = 128
D_MODEL = 1024
WINDOW = 128
SCALE = 0.08838834764831843
HALF = 512


def kernel(x, Wq, K_ext, V_ext, Wo):
    def body(x_ref, wq_ref, k_hbm, v_hbm, wo_ref, out_ref,
             k_vmem, v_vmem, kb_ref, vb_ref, wob_ref, q_ref, ctx_ref,
             arA, arB, comm1_ref, comm2_ref,
             send_sems, recv_sems, cp_sems):
        my = lax.axis_index("i")
        head0 = my * H_LOC

        p1 = my ^ 1
        p2 = 3 - my
        h1 = ((my == 1) | (my == 2)).astype(jnp.int32)
        h2 = my // 2
        g1 = my // 2
        g2 = my % 2
        a_keep, a_send = h1 * 256, (1 - h1) * 256
        b_keep, b_send = g1 * 256, (1 - g1) * 256
        qa = a_keep + h2 * 128
        qb = b_keep + g2 * 128

        barrier = pltpu.get_barrier_semaphore()
        for nbr in [p1, p2]:
            pl.semaphore_signal(
                barrier, inc=1,
                device_id=(nbr,), device_id_type=pl.DeviceIdType.MESH)
        pl.semaphore_wait(barrier, 2)

        cp_k = pltpu.make_async_copy(
            k_hbm.at[0, :, pl.ds(head0, H_LOC), :], k_vmem, cp_sems.at[0])
        cp_v = pltpu.make_async_copy(
            v_hbm.at[0, :, pl.ds(head0, H_LOC), :], v_vmem, cp_sems.at[1])
        cp_k.start()
        cp_v.start()

        q_ref[...] = jnp.dot(
            x_ref[0].astype(jnp.bfloat16), wq_ref[...].astype(jnp.bfloat16),
            preferred_element_type=jnp.float32).astype(jnp.bfloat16)
        wob_ref[...] = wo_ref[...].astype(jnp.bfloat16)

        cp_k.wait()
        kb_ref[...] = k_vmem[...].astype(jnp.bfloat16)
        cp_v.wait()
        vb_ref[...] = v_vmem[...].astype(jnp.bfloat16)

        neg = jnp.float32(-1e9)
        iota_q = lax.broadcasted_iota(jnp.int32, (HALF, SKV), 0)
        iota_k = lax.broadcasted_iota(jnp.int32, (HALF, SKV), 1)

        def compute_half(r0, dst):
            mask = jnp.abs(iota_q + r0 - iota_k) <= WINDOW
            for h in range(H_LOC):
                qh = q_ref[r0:r0 + HALF, h * DH:(h + 1) * DH]
                s = lax.dot_general(
                    qh, kb_ref[:, h, :], (((1,), (1,)), ((), ())),
                    preferred_element_type=jnp.float32) * SCALE
                s = jnp.where(mask, s, neg)
                m = jnp.max(s, axis=1, keepdims=True)
                w = jnp.exp(s - m)
                wb = (w * (1.0 / jnp.sum(w, axis=1, keepdims=True))
                      ).astype(jnp.bfloat16)
                ctx_ref[r0:r0 + HALF, h * DH:(h + 1) * DH] = jnp.dot(
                    wb, vb_ref[:, h, :],
                    preferred_element_type=jnp.float32).astype(jnp.bfloat16)
            dst[...] = jnp.dot(
                ctx_ref[r0:r0 + HALF, :], wob_ref[...],
                preferred_element_type=jnp.float32).astype(jnp.bfloat16)

        def exch(src_ref, src_rows, n_rows, dst_ref, dst_rows, sem_idx, peer):
            return pltpu.make_async_remote_copy(
                src_ref=src_ref.at[pl.ds(src_rows, n_rows), :],
                dst_ref=dst_ref.at[pl.ds(dst_rows, n_rows), :],
                send_sem=send_sems.at[sem_idx],
                recv_sem=recv_sems.at[sem_idx],
                device_id=(peer,),
                device_id_type=pl.DeviceIdType.MESH)

        compute_half(0, arA)
        ra = exch(arA, a_send, 256, comm1_ref, 0, 0, p1)
        ra.start()
        compute_half(HALF, arB)
        rb = exch(arB, b_send, 256, comm1_ref, 256, 1, p2)
        rb.start()

        ra.wait()
        rows = pl.ds(a_keep, 256)
        arA[rows, :] = arA[rows, :] + comm1_ref[pl.ds(0, 256), :]
        ra = exch(arA, a_keep + (1 - h2) * 128, 128, comm2_ref, 0, 2, p2)
        ra.start()
        rb.wait()
        rows = pl.ds(b_keep, 256)
        arB[rows, :] = arB[rows, :] + comm1_ref[pl.ds(256, 256), :]
        rb = exch(arB, b_keep + (1 - g2) * 128, 128, comm2_ref, 128, 3, p1)
        rb.start()

        ra.wait()
        rows = pl.ds(qa, 128)
        arA[rows, :] = arA[rows, :] + comm2_ref[pl.ds(0, 128), :]
        ra = exch(arA, qa, 128, arA, qa, 4, p2)
        ra.start()
        rb.wait()
        rows = pl.ds(qb, 128)
        arB[rows, :] = arB[rows, :] + comm2_ref[pl.ds(128, 128), :]
        rb = exch(arB, qb, 128, arB, qb, 5, p1)
        rb.start()

        ra.wait()
        ra = exch(arA, a_keep, 256, arA, a_keep, 6, p1)
        ra.start()
        rb.wait()
        rb = exch(arB, b_keep, 256, arB, b_keep, 7, p2)
        rb.start()
        ra.wait()
        rb.wait()

        out_ref[0, 0:HALF, :] = arA[...].astype(jnp.float32)
        out_ref[0, HALF:SQ, :] = arB[...].astype(jnp.float32)

    return pl.pallas_call(
        body,
        out_shape=jax.ShapeDtypeStruct((1, SQ, D_MODEL), jnp.float32),
        in_specs=[
            pl.BlockSpec(memory_space=pltpu.VMEM),
            pl.BlockSpec(memory_space=pltpu.VMEM),
            pl.BlockSpec(memory_space=pl.ANY),
            pl.BlockSpec(memory_space=pl.ANY),
            pl.BlockSpec(memory_space=pltpu.VMEM),
        ],
        out_specs=pl.BlockSpec(memory_space=pltpu.VMEM),
        scratch_shapes=[
            pltpu.VMEM((SKV, H_LOC, DH), jnp.float32),
            pltpu.VMEM((SKV, H_LOC, DH), jnp.float32),
            pltpu.VMEM((SKV, H_LOC, DH), jnp.bfloat16),
            pltpu.VMEM((SKV, H_LOC, DH), jnp.bfloat16),
            pltpu.VMEM((D_MODEL, D_MODEL), jnp.bfloat16),
            pltpu.VMEM((SQ, D_MODEL), jnp.bfloat16),
            pltpu.VMEM((SQ, D_MODEL), jnp.bfloat16),
            pltpu.VMEM((HALF, D_MODEL), jnp.bfloat16),
            pltpu.VMEM((HALF, D_MODEL), jnp.bfloat16),
            pltpu.VMEM((512, D_MODEL), jnp.bfloat16),
            pltpu.VMEM((256, D_MODEL), jnp.bfloat16),
            pltpu.SemaphoreType.DMA((8,)),
            pltpu.SemaphoreType.DMA((8,)),
            pltpu.SemaphoreType.DMA((2,)),
        ],
        compiler_params=pltpu.CompilerParams(collective_id=0),
    )(x, Wq, K_ext, V_ext, Wo)
